# Initial kernel scaffold; baseline (speedup 1.0000x reference)
import functools

import jax
import jax.numpy as jnp
from jax import lax
from jax.experimental import pallas as pl
from jax.experimental.pallas import tpu as pltpu
from jax.experimental.pallas import tpu_sc as plsc

N = 10000
E = 320000
D = 128

NC = 2
NS = 16
NW = NC * NS
WIN = 128
PT = 80
ROWS_PAD = NW * PT
E_PAD = ROWS_PAD * WIN
NPAD = 16
NACC = N + NPAD
ZPT = NACC // NS
RPT = N // NS

_mesh = plsc.VectorSubcoreMesh(core_axis_name="c", subcore_axis_name="s")


@functools.partial(
    pl.kernel,
    out_type=jax.ShapeDtypeStruct((NC, N, 16), jnp.float32),
    mesh=_mesh,
    scratch_types=[
        pltpu.VMEM((PT, WIN), jnp.int32),
        pltpu.VMEM((WIN, 16), jnp.float32),
        pltpu.VMEM_SHARED((NACC, 16), jnp.float32),
    ],
)
def _deg_kernel(dst_hbm, out_hbm, idst_v, ones_v, acc):
    cid = lax.axis_index("c")
    sid = lax.axis_index("s")
    wid = cid * NS + sid

    @pl.loop(0, WIN)
    def _(j):
        ones_v[j, :] = jnp.zeros((16,), jnp.float32)

    zbase = sid * ZPT

    @pl.loop(0, 4)
    def _(k):
        pltpu.sync_copy(ones_v, acc.at[pl.ds(zbase + k * WIN, WIN)])

    pltpu.sync_copy(ones_v.at[pl.ds(0, ZPT - 4 * WIN)],
                    acc.at[pl.ds(zbase + 4 * WIN, ZPT - 4 * WIN)])

    @pl.loop(0, WIN)
    def _(j):
        ones_v[j, :] = jnp.full((16,), 1.0, jnp.float32)

    plsc.subcore_barrier()

    pltpu.sync_copy(dst_hbm.at[pl.ds(wid * PT, PT)], idst_v)

    @pl.loop(0, PT)
    def _(j):
        pltpu.sync_copy(ones_v, acc.at[idst_v.at[j]], add=True)

    plsc.subcore_barrier()
    pltpu.sync_copy(acc.at[pl.ds(sid * RPT, RPT)],
                    out_hbm.at[cid].at[pl.ds(sid * RPT, RPT)])


@functools.partial(
    pl.kernel,
    out_type=jax.ShapeDtypeStruct((NC, N, D), jnp.float32),
    mesh=_mesh,
    scratch_types=[
        pltpu.VMEM((PT, WIN), jnp.int32),
        pltpu.VMEM((PT, WIN), jnp.int32),
        pltpu.VMEM((WIN, D), jnp.float32),
        pltpu.VMEM_SHARED((NACC, D), jnp.float32),
    ],
)
def _agg_kernel(h_hbm, src_hbm, dst_hbm, out_hbm, isrc_v, idst_v, rows_v, acc):
    cid = lax.axis_index("c")
    sid = lax.axis_index("s")
    wid = cid * NS + sid

    @pl.loop(0, WIN)
    def _(j):
        @pl.loop(0, D, step=16)
        def _(c):
            rows_v[j, pl.ds(c, 16)] = jnp.zeros((16,), jnp.float32)

    zbase = sid * ZPT

    @pl.loop(0, 4)
    def _(k):
        pltpu.sync_copy(rows_v, acc.at[pl.ds(zbase + k * WIN, WIN)])

    pltpu.sync_copy(rows_v.at[pl.ds(0, ZPT - 4 * WIN)],
                    acc.at[pl.ds(zbase + 4 * WIN, ZPT - 4 * WIN)])

    plsc.subcore_barrier()

    pltpu.sync_copy(src_hbm.at[pl.ds(wid * PT, PT)], isrc_v)
    pltpu.sync_copy(dst_hbm.at[pl.ds(wid * PT, PT)], idst_v)

    @pl.loop(0, PT)
    def _(j):
        pltpu.sync_copy(h_hbm.at[isrc_v.at[j]], rows_v)
        pltpu.sync_copy(rows_v, acc.at[idst_v.at[j]], add=True)

    plsc.subcore_barrier()
    pltpu.sync_copy(acc.at[pl.ds(sid * RPT, RPT)],
                    out_hbm.at[cid].at[pl.ds(sid * RPT, RPT)])


BM = 2000


def _mm_body(x_ref, w_ref, o_ref):
    o_ref[...] = jnp.dot(x_ref[...], w_ref[...],
                         preferred_element_type=jnp.float32)


def _matmul(x, w):
    return pl.pallas_call(
        _mm_body,
        grid=(N // BM,),
        in_specs=[pl.BlockSpec((BM, D), lambda i: (i, 0)),
                  pl.BlockSpec((D, D), lambda i: (0, 0))],
        out_specs=pl.BlockSpec((BM, D), lambda i: (i, 0)),
        out_shape=jax.ShapeDtypeStruct((N, D), jnp.float32),
    )(x, w)


def _scale_body(dp_ref, h_ref, dinv_ref, hp_ref):
    d = dp_ref[0, :, 0] + dp_ref[1, :, 0] + 1.0
    dinv = lax.rsqrt(d)[:, None]
    dinv_ref[...] = jnp.broadcast_to(dinv, dinv_ref.shape)
    hp_ref[...] = h_ref[...] * dinv


def _scale(dparts, h):
    return pl.pallas_call(
        _scale_body,
        grid=(N // BM,),
        in_specs=[pl.BlockSpec((NC, BM, 16), lambda i: (0, i, 0)),
                  pl.BlockSpec((BM, D), lambda i: (i, 0))],
        out_specs=[pl.BlockSpec((BM, D), lambda i: (i, 0)),
                   pl.BlockSpec((BM, D), lambda i: (i, 0))],
        out_shape=[jax.ShapeDtypeStruct((N, D), jnp.float32),
                   jax.ShapeDtypeStruct((N, D), jnp.float32)],
    )(dparts, h)


def _mid_body(s_ref, hp_ref, dinv_ref, b_ref, w_ref, o_ref):
    t = (s_ref[0] + s_ref[1] + hp_ref[...]) * dinv_ref[...] + b_ref[...]
    u = jnp.maximum(t, 0.0)
    o_ref[...] = jnp.dot(u, w_ref[...],
                         preferred_element_type=jnp.float32) * dinv_ref[...]


def _mid(s1, h1p, dinv_b, b1, w2):
    return pl.pallas_call(
        _mid_body,
        grid=(N // BM,),
        in_specs=[pl.BlockSpec((NC, BM, D), lambda i: (0, i, 0)),
                  pl.BlockSpec((BM, D), lambda i: (i, 0)),
                  pl.BlockSpec((BM, D), lambda i: (i, 0)),
                  pl.BlockSpec((1, D), lambda i: (0, 0)),
                  pl.BlockSpec((D, D), lambda i: (0, 0))],
        out_specs=pl.BlockSpec((BM, D), lambda i: (i, 0)),
        out_shape=jax.ShapeDtypeStruct((N, D), jnp.float32),
    )(s1, h1p, dinv_b, b1, w2)


def _fin_body(s_ref, hp_ref, dinv_ref, b_ref, o_ref):
    o_ref[...] = ((s_ref[0] + s_ref[1] + hp_ref[...]) * dinv_ref[...]
                  + b_ref[...])


def _fin(s2, h2p, dinv_b, b2):
    return pl.pallas_call(
        _fin_body,
        grid=(N // BM,),
        in_specs=[pl.BlockSpec((NC, BM, D), lambda i: (0, i, 0)),
                  pl.BlockSpec((BM, D), lambda i: (i, 0)),
                  pl.BlockSpec((BM, D), lambda i: (i, 0)),
                  pl.BlockSpec((1, D), lambda i: (0, 0))],
        out_specs=pl.BlockSpec((BM, D), lambda i: (i, 0)),
        out_shape=jax.ShapeDtypeStruct((N, D), jnp.float32),
    )(s2, h2p, dinv_b, b2)


def kernel(x, pos_edge_index, W1, b1, W2, b2):
    src = pos_edge_index[0]
    dst = pos_edge_index[1]
    pad = jnp.arange(E_PAD - E, dtype=pos_edge_index.dtype)
    srcp = jnp.concatenate([src, pad % 4096]).reshape(ROWS_PAD, WIN)
    dstp = jnp.concatenate([dst, N + (pad % NPAD)]).reshape(ROWS_PAD, WIN)

    dparts = _deg_kernel(dstp)
    h1 = _matmul(x, W1)
    dinv_b, h1p = _scale(dparts, h1)
    s1 = _agg_kernel(h1p, srcp, dstp)
    h2p = _mid(s1, h1p, dinv_b, b1.reshape(1, D), W2)
    s2 = _agg_kernel(h2p, srcp, dstp)
    return _fin(s2, h2p, dinv_b, b2.reshape(1, D))


# speedup vs baseline: 22.8297x; 22.8297x over previous
import functools

import jax
import jax.numpy as jnp
from jax import lax
from jax.experimental import pallas as pl
from jax.experimental.pallas import tpu as pltpu
from jax.experimental.pallas import tpu_sc as plsc

N = 10000
E = 320000
D = 128

NC = 2
NS = 16
NW = NC * NS
WIN = 128
PT = 80
ROWS_PAD = NW * PT
E_PAD = ROWS_PAD * WIN
NPAD = 112
NACC = N + NPAD
ZPT = NACC // NS

_mesh = plsc.VectorSubcoreMesh(core_axis_name="c", subcore_axis_name="s")


@functools.partial(
    pl.kernel,
    out_type=jax.ShapeDtypeStruct((NC, NACC, 16), jnp.float32),
    mesh=_mesh,
    scratch_types=[
        pltpu.VMEM((PT, WIN), jnp.int32),
        pltpu.VMEM((WIN, 16), jnp.float32),
        pltpu.VMEM_SHARED((NACC, 16), jnp.float32),
    ],
)
def _deg_kernel(dst_hbm, out_hbm, idst_v, ones_v, acc):
    cid = lax.axis_index("c")
    sid = lax.axis_index("s")
    wid = cid * NS + sid

    @pl.loop(0, WIN)
    def _(j):
        ones_v[j, :] = jnp.zeros((16,), jnp.float32)

    zbase = sid * ZPT

    @pl.loop(0, 4)
    def _(k):
        pltpu.sync_copy(ones_v, acc.at[pl.ds(zbase + k * WIN, WIN)])

    pltpu.sync_copy(ones_v.at[pl.ds(0, ZPT - 4 * WIN)],
                    acc.at[pl.ds(zbase + 4 * WIN, ZPT - 4 * WIN)])

    @pl.loop(0, WIN)
    def _(j):
        ones_v[j, :] = jnp.full((16,), 1.0, jnp.float32)

    plsc.subcore_barrier()

    pltpu.sync_copy(dst_hbm.at[pl.ds(wid * PT, PT)], idst_v)

    @pl.loop(0, PT)
    def _(j):
        pltpu.sync_copy(ones_v, acc.at[idst_v.at[j]], add=True)

    plsc.subcore_barrier()
    pltpu.sync_copy(acc.at[pl.ds(sid * ZPT, ZPT)],
                    out_hbm.at[cid].at[pl.ds(sid * ZPT, ZPT)])


@functools.partial(
    pl.kernel,
    out_type=jax.ShapeDtypeStruct((NC, NACC, D), jnp.float32),
    mesh=_mesh,
    scratch_types=[
        pltpu.VMEM((PT, WIN), jnp.int32),
        pltpu.VMEM((PT, WIN), jnp.int32),
        pltpu.VMEM((WIN, D), jnp.float32),
        pltpu.VMEM_SHARED((NACC, D), jnp.float32),
    ],
)
def _agg_kernel(h_hbm, src_hbm, dst_hbm, out_hbm, isrc_v, idst_v, rows_v, acc):
    cid = lax.axis_index("c")
    sid = lax.axis_index("s")
    wid = cid * NS + sid

    @pl.loop(0, WIN)
    def _(j):
        @pl.loop(0, D, step=16)
        def _(c):
            rows_v[j, pl.ds(c, 16)] = jnp.zeros((16,), jnp.float32)

    zbase = sid * ZPT

    @pl.loop(0, 4)
    def _(k):
        pltpu.sync_copy(rows_v, acc.at[pl.ds(zbase + k * WIN, WIN)])

    pltpu.sync_copy(rows_v.at[pl.ds(0, ZPT - 4 * WIN)],
                    acc.at[pl.ds(zbase + 4 * WIN, ZPT - 4 * WIN)])

    plsc.subcore_barrier()

    pltpu.sync_copy(src_hbm.at[pl.ds(wid * PT, PT)], isrc_v)
    pltpu.sync_copy(dst_hbm.at[pl.ds(wid * PT, PT)], idst_v)

    @pl.loop(0, PT)
    def _(j):
        pltpu.sync_copy(h_hbm.at[isrc_v.at[j]], rows_v)
        pltpu.sync_copy(rows_v, acc.at[idst_v.at[j]], add=True)

    plsc.subcore_barrier()
    pltpu.sync_copy(acc.at[pl.ds(sid * ZPT, ZPT)],
                    out_hbm.at[cid].at[pl.ds(sid * ZPT, ZPT)])


BM = 2000


def _mm_body(x_ref, w_ref, o_ref):
    o_ref[...] = jnp.dot(x_ref[...], w_ref[...],
                         preferred_element_type=jnp.float32)


def _matmul(x, w):
    return pl.pallas_call(
        _mm_body,
        grid=(N // BM,),
        in_specs=[pl.BlockSpec((BM, D), lambda i: (i, 0)),
                  pl.BlockSpec((D, D), lambda i: (0, 0))],
        out_specs=pl.BlockSpec((BM, D), lambda i: (i, 0)),
        out_shape=jax.ShapeDtypeStruct((N, D), jnp.float32),
    )(x, w)


def _scale_body(dp_ref, h_ref, dinv_ref, hp_ref):
    d = dp_ref[0, :, 0] + dp_ref[1, :, 0] + 1.0
    dinv = lax.rsqrt(d)[:, None]
    dinv_ref[...] = jnp.broadcast_to(dinv, dinv_ref.shape)
    hp_ref[...] = h_ref[...] * dinv


def _scale(dparts, h):
    return pl.pallas_call(
        _scale_body,
        grid=(N // BM,),
        in_specs=[pl.BlockSpec((NC, BM, 16), lambda i: (0, i, 0)),
                  pl.BlockSpec((BM, D), lambda i: (i, 0))],
        out_specs=[pl.BlockSpec((BM, D), lambda i: (i, 0)),
                   pl.BlockSpec((BM, D), lambda i: (i, 0))],
        out_shape=[jax.ShapeDtypeStruct((N, D), jnp.float32),
                   jax.ShapeDtypeStruct((N, D), jnp.float32)],
    )(dparts, h)


def _mid_body(s_ref, hp_ref, dinv_ref, b_ref, w_ref, o_ref):
    t = (s_ref[0] + s_ref[1] + hp_ref[...]) * dinv_ref[...] + b_ref[...]
    u = jnp.maximum(t, 0.0)
    o_ref[...] = jnp.dot(u, w_ref[...],
                         preferred_element_type=jnp.float32) * dinv_ref[...]


def _mid(s1, h1p, dinv_b, b1, w2):
    return pl.pallas_call(
        _mid_body,
        grid=(N // BM,),
        in_specs=[pl.BlockSpec((NC, BM, D), lambda i: (0, i, 0)),
                  pl.BlockSpec((BM, D), lambda i: (i, 0)),
                  pl.BlockSpec((BM, D), lambda i: (i, 0)),
                  pl.BlockSpec((1, D), lambda i: (0, 0)),
                  pl.BlockSpec((D, D), lambda i: (0, 0))],
        out_specs=pl.BlockSpec((BM, D), lambda i: (i, 0)),
        out_shape=jax.ShapeDtypeStruct((N, D), jnp.float32),
    )(s1, h1p, dinv_b, b1, w2)


def _fin_body(s_ref, hp_ref, dinv_ref, b_ref, o_ref):
    o_ref[...] = ((s_ref[0] + s_ref[1] + hp_ref[...]) * dinv_ref[...]
                  + b_ref[...])


def _fin(s2, h2p, dinv_b, b2):
    return pl.pallas_call(
        _fin_body,
        grid=(N // BM,),
        in_specs=[pl.BlockSpec((NC, BM, D), lambda i: (0, i, 0)),
                  pl.BlockSpec((BM, D), lambda i: (i, 0)),
                  pl.BlockSpec((BM, D), lambda i: (i, 0)),
                  pl.BlockSpec((1, D), lambda i: (0, 0))],
        out_specs=pl.BlockSpec((BM, D), lambda i: (i, 0)),
        out_shape=jax.ShapeDtypeStruct((N, D), jnp.float32),
    )(s2, h2p, dinv_b, b2)


def kernel(x, pos_edge_index, W1, b1, W2, b2):
    src = pos_edge_index[0]
    dst = pos_edge_index[1]
    pad = jnp.arange(E_PAD - E, dtype=pos_edge_index.dtype)
    srcp = jnp.concatenate([src, pad % 4096]).reshape(ROWS_PAD, WIN)
    dstp = jnp.concatenate([dst, N + (pad % NPAD)]).reshape(ROWS_PAD, WIN)

    dparts = _deg_kernel(dstp)
    h1 = _matmul(x, W1)
    dinv_b, h1p = _scale(dparts, h1)
    s1 = _agg_kernel(h1p, srcp, dstp)
    h2p = _mid(s1, h1p, dinv_b, b1.reshape(1, D), W2)
    s2 = _agg_kernel(h2p, srcp, dstp)
    return _fin(s2, h2p, dinv_b, b2.reshape(1, D))
